# Initial kernel scaffold; baseline (speedup 1.0000x reference)
import jax
import jax.numpy as jnp
from jax import lax
from jax.experimental import pallas as pl
from jax.experimental.pallas import tpu as pltpu
from jax.experimental.pallas import tpu_sc as plsc

D_MODEL = 32
SCALE = float(D_MODEL) ** 0.5

B, T = 4096, 200
N = B * T
NUM_WORKERS = 32
PER_W = N // NUM_WORKERS
CHUNK = 1600
NCHUNK = PER_W // CHUNK


def _body(x_hbm, table_hbm, out_hbm, idx_v, rows_v, sem):
    wid = lax.axis_index("s") * 2 + lax.axis_index("c")
    base = wid * PER_W

    @pl.loop(0, NCHUNK)
    def _chunk(ci):
        off = base + ci * CHUNK
        pltpu.sync_copy(x_hbm.at[pl.ds(off, CHUNK)], idx_v)
        pltpu.async_copy(table_hbm.at[idx_v], rows_v, sem).wait()

        @pl.loop(0, CHUNK, unroll=8)
        def _scale(j):
            rows_v[j, 0:16] = rows_v[j, 0:16] * SCALE
            rows_v[j, 16:32] = rows_v[j, 16:32] * SCALE

        pltpu.sync_copy(rows_v, out_hbm.at[pl.ds(off, CHUNK)])


def kernel(x, table):
    xf = x.reshape(N).astype(jnp.int32)
    mesh = plsc.VectorSubcoreMesh(core_axis_name="c", subcore_axis_name="s")
    out = pl.kernel(
        _body,
        out_type=jax.ShapeDtypeStruct((N, D_MODEL), jnp.float32),
        mesh=mesh,
        scratch_types=[
            pltpu.VMEM((CHUNK,), jnp.int32),
            pltpu.VMEM((CHUNK, D_MODEL), jnp.float32),
            pltpu.SemaphoreType.DMA,
        ],
    )(xf, table)
    return out.reshape(B, T, D_MODEL)


# speedup vs baseline: 1.4169x; 1.4169x over previous
import jax
import jax.numpy as jnp
from jax import lax
from jax.experimental import pallas as pl
from jax.experimental.pallas import tpu as pltpu
from jax.experimental.pallas import tpu_sc as plsc

D_MODEL = 32
SCALE = float(D_MODEL) ** 0.5

B, T = 4096, 200
N = B * T
NUM_WORKERS = 32
PER_W = N // NUM_WORKERS
CHUNK = 1600
NCHUNK = PER_W // CHUNK


def _body(x_hbm, table_hbm, out_hbm, idx_v, rows_v, sem):
    wid = lax.axis_index("s") * 2 + lax.axis_index("c")
    base = wid * PER_W

    @pl.loop(0, NCHUNK)
    def _chunk(ci):
        off = base + ci * CHUNK
        pltpu.sync_copy(x_hbm.at[pl.ds(off, CHUNK)], idx_v)
        pltpu.async_copy(table_hbm.at[idx_v], rows_v, sem).wait()

        @pl.loop(0, CHUNK, unroll=8)
        def _scale(j):
            rows_v[j, 0:16] = rows_v[j, 0:16] * SCALE
            rows_v[j, 16:32] = rows_v[j, 16:32] * SCALE

        pltpu.sync_copy(rows_v, out_hbm.at[pl.ds(off, CHUNK)])


def kernel(x, table):
    xf = x.reshape(N).astype(jnp.int32)
    mesh = plsc.VectorSubcoreMesh(core_axis_name="c", subcore_axis_name="s")
    out = pl.kernel(
        _body,
        out_type=jax.ShapeDtypeStruct((N, D_MODEL), jnp.float32),
        mesh=mesh,
        compiler_params=pltpu.CompilerParams(use_tc_tiling_on_sc=False),
        scratch_types=[
            pltpu.VMEM((CHUNK,), jnp.int32),
            pltpu.VMEM((CHUNK, D_MODEL), jnp.float32),
            pltpu.SemaphoreType.DMA,
        ],
    )(xf, table)
    return out.reshape(B, T, D_MODEL)


# speedup vs baseline: 1.4708x; 1.0380x over previous
import jax
import jax.numpy as jnp
from jax import lax
from jax.experimental import pallas as pl
from jax.experimental.pallas import tpu as pltpu
from jax.experimental.pallas import tpu_sc as plsc

D_MODEL = 32
SCALE = float(D_MODEL) ** 0.5

B, T = 4096, 200
N = B * T
NUM_WORKERS = 32
PER_W = N // NUM_WORKERS
NBUF = 4
CHUNK = 800
NCHUNK = PER_W // CHUNK
NBLK = NCHUNK // NBUF


def _body(x_hbm, table_hbm, out_hbm, idx_all,
          r0, r1, r2, r3, g0, g1, g2, g3, o0, o1, o2, o3):
    rows = (r0, r1, r2, r3)
    sg = (g0, g1, g2, g3)
    so = (o0, o1, o2, o3)

    wid = lax.axis_index("s") * 2 + lax.axis_index("c")
    base = wid * PER_W
    pltpu.sync_copy(x_hbm.at[pl.ds(base, PER_W)], idx_all)

    def gather(c, b):
        return pltpu.async_copy(
            table_hbm.at[idx_all.at[pl.ds(c * CHUNK, CHUNK)]], rows[b], sg[b])

    def write_out(c, b):
        return pltpu.async_copy(
            rows[b], out_hbm.at[pl.ds(base + c * CHUNK, CHUNK)], so[b])

    gd = {}
    od = {}
    for b in range(NBUF):
        gd[b] = gather(b, b)

    for blk in range(NBLK):
        for b in range(NBUF):
            c = blk * NBUF + b
            gd[b].wait()
            rb = rows[b]

            @pl.loop(0, CHUNK, unroll=8)
            def _scale(j):
                rb[j, 0:16] = rb[j, 0:16] * SCALE
                rb[j, 16:32] = rb[j, 16:32] * SCALE

            od[b] = write_out(c, b)
        for b in range(NBUF):
            cn = blk * NBUF + b + NBUF
            if cn < NCHUNK:
                od[b].wait()
                gd[b] = gather(cn, b)

    for b in range(NBUF):
        od[b].wait()


def kernel(x, table):
    xf = x.reshape(N).astype(jnp.int32)
    mesh = plsc.VectorSubcoreMesh(core_axis_name="c", subcore_axis_name="s")
    out = pl.kernel(
        _body,
        out_type=jax.ShapeDtypeStruct((N, D_MODEL), jnp.float32),
        mesh=mesh,
        compiler_params=pltpu.CompilerParams(use_tc_tiling_on_sc=False),
        scratch_types=(
            [pltpu.VMEM((PER_W,), jnp.int32)]
            + [pltpu.VMEM((CHUNK, D_MODEL), jnp.float32) for _ in range(NBUF)]
            + [pltpu.SemaphoreType.DMA for _ in range(2 * NBUF)]
        ),
    )(xf, table)
    return out.reshape(B, T, D_MODEL)
